# baseline (speedup 1.0000x reference)
import functools

import jax
import jax.numpy as jnp
from jax import lax
from jax.experimental import pallas as pl
from jax.experimental.pallas import tpu as pltpu
from jax.experimental.pallas import tpu_sc as plsc

_OBS_VOCAB = 1000000
_NUM_ACTIONS = 8
_ACTION_DIM = 16
_EMBED_DIM = 64
_B = 1024
_L = 200
_SEQ = _L + _NUM_ACTIONS

_NC, _NS, _LANES = 2, 16, 16
_NW = _NC * _NS
_B_PER_W = _B // _NW
_ACT_PER_W = _B_PER_W * _NUM_ACTIONS

_TOK_SPLITS = ((0, 128), (128, 72))


def _body(tokens_hbm, action_hbm, obs_hbm, atab_hbm, out_hbm,
          idx_v, act_idx_v, rows_v, sem):
    cid = lax.axis_index("c")
    sid = lax.axis_index("s")
    wid = sid * _NC + cid
    b0 = wid * _B_PER_W

    pltpu.sync_copy(action_hbm.at[pl.ds(b0 * _NUM_ACTIONS, _ACT_PER_W)],
                    act_idx_v)
    for r in range(_ACT_PER_W // _LANES):
        flat = r * _LANES + lax.iota(jnp.int32, _LANES)
        cols = lax.bitwise_and(flat, _NUM_ACTIONS - 1)
        sl = pl.ds(r * _LANES, _LANES)
        act_idx_v[sl] = act_idx_v[sl] + cols * _ACTION_DIM

    @pl.loop(0, _B_PER_W)
    def _batch_loop(j):
        b = b0 + j
        pltpu.sync_copy(tokens_hbm.at[b], idx_v)
        copies = []
        for off, width in _TOK_SPLITS:
            copies.append(pltpu.async_copy(
                obs_hbm.at[idx_v.at[pl.ds(off, width)]],
                rows_v.at[pl.ds(off, width)], sem))
        copies.append(pltpu.async_copy(
            atab_hbm.at[act_idx_v.at[pl.ds(j * _NUM_ACTIONS, _NUM_ACTIONS)]],
            rows_v.at[pl.ds(_L, _NUM_ACTIONS)], sem))
        for cp in copies:
            cp.wait()
        pltpu.sync_copy(rows_v, out_hbm.at[b])


@jax.jit
def kernel(tokens, action, obs_table, action_table):
    mesh = plsc.VectorSubcoreMesh(core_axis_name="c", subcore_axis_name="s")
    run = pl.kernel(
        _body,
        out_type=jax.ShapeDtypeStruct((_B, _SEQ, _EMBED_DIM), jnp.float32),
        mesh=mesh,
        scratch_types=[
            pltpu.VMEM((_L,), jnp.int32),
            pltpu.VMEM((_ACT_PER_W,), jnp.int32),
            pltpu.VMEM((_SEQ, _EMBED_DIM), jnp.float32),
            pltpu.SemaphoreType.DMA,
        ],
        compiler_params=pltpu.CompilerParams(use_tc_tiling_on_sc=False),
    )
    return run(tokens.astype(jnp.int32),
               action.astype(jnp.int32).reshape(-1),
               obs_table, action_table)


# speedup vs baseline: 1.0501x; 1.0501x over previous
import jax
import jax.numpy as jnp
from jax import lax
from jax.experimental import pallas as pl
from jax.experimental.pallas import tpu as pltpu
from jax.experimental.pallas import tpu_sc as plsc

_NUM_ACTIONS = 8
_ACTION_DIM = 16
_EMBED_DIM = 64
_B = 1024
_L = 200
_SEQ = _L + _NUM_ACTIONS

_NC, _NS, _LANES = 2, 16, 16
_NW = _NC * _NS
_B_PER_W = _B // _NW
_ACT_PER_W = _B_PER_W * _NUM_ACTIONS

_TOK_SPLITS = ((0, 128), (128, 72))

_R = 8
_A = 4


def _gather_descs(j, jb, tok_v, act_idx_v, obs_hbm, atab_hbm, rows, gsem):
    d = []
    for off, width in _TOK_SPLITS:
        d.append(pltpu.make_async_copy(
            obs_hbm.at[tok_v.at[j, pl.ds(off, width)]],
            rows.at[pl.ds(off, width)], gsem))
    d.append(pltpu.make_async_copy(
        atab_hbm.at[act_idx_v.at[pl.ds(jb, _NUM_ACTIONS)]],
        rows.at[pl.ds(_L, _NUM_ACTIONS)], gsem))
    return d


def _body(tokens_hbm, action_hbm, obs_hbm, atab_hbm, out_hbm,
          tok_v, act_idx_v, *rest):
    rows = rest[:_R]
    gsems = rest[_R:2 * _R]
    ssems = rest[2 * _R:3 * _R]

    cid = lax.axis_index("c")
    sid = lax.axis_index("s")
    wid = sid * _NC + cid
    b0 = wid * _B_PER_W

    pltpu.sync_copy(tokens_hbm.at[pl.ds(b0, _B_PER_W)], tok_v)
    pltpu.sync_copy(action_hbm.at[pl.ds(b0 * _NUM_ACTIONS, _ACT_PER_W)],
                    act_idx_v)

    for r in range(_ACT_PER_W // _LANES):
        flat = r * _LANES + lax.iota(jnp.int32, _LANES)
        cols = lax.bitwise_and(flat, _NUM_ACTIONS - 1)
        sl = pl.ds(r * _LANES, _LANES)
        act_idx_v[sl] = act_idx_v[sl] + cols * _ACTION_DIM

    def gathers(j, jb, buf):
        return _gather_descs(j, jb, tok_v, act_idx_v, obs_hbm, atab_hbm,
                             rows[buf], gsems[buf])

    def store(j, buf):
        return pltpu.make_async_copy(rows[buf], out_hbm.at[b0 + j],
                                     ssems[buf])

    for u in range(_A):
        for dsc in gathers(u, u * _NUM_ACTIONS, u):
            dsc.start()

    @pl.loop(0, _B_PER_W // _R)
    def _grp(i):
        for u in range(_R):
            j = i * _R + u
            for dsc in gathers(j, j * _NUM_ACTIONS, u):
                dsc.wait()
            store(j, u).start()
            jn = j + _A
            bn = (u + _A) % _R

            @pl.when(jn < _B_PER_W)
            def _issue_next():
                @pl.when(j >= _R - _A)
                def _drain_store():
                    store(j - (_R - _A), bn).wait()
                for dsc in gathers(jn, jn * _NUM_ACTIONS, bn):
                    dsc.start()

    for u in range(_R):
        store(_B_PER_W - _R + u, u).wait()


@jax.jit
def kernel(tokens, action, obs_table, action_table):
    mesh = plsc.VectorSubcoreMesh(core_axis_name="c", subcore_axis_name="s")
    run = pl.kernel(
        _body,
        out_type=jax.ShapeDtypeStruct((_B, _SEQ, _EMBED_DIM), jnp.float32),
        mesh=mesh,
        scratch_types=[
            pltpu.VMEM((_B_PER_W, _L), jnp.int32),
            pltpu.VMEM((_ACT_PER_W,), jnp.int32),
            *[pltpu.VMEM((_SEQ, _EMBED_DIM), jnp.float32) for _ in range(_R)],
            *[pltpu.SemaphoreType.DMA for _ in range(2 * _R)],
        ],
        compiler_params=pltpu.CompilerParams(use_tc_tiling_on_sc=False),
    )
    return run(tokens.astype(jnp.int32),
               action.astype(jnp.int32).reshape(-1),
               obs_table, action_table)
